# Initial kernel scaffold; baseline (speedup 1.0000x reference)
import jax
import jax.numpy as jnp
from jax.experimental import pallas as pl


def kernel(x, W, b):
    raise NotImplementedError("write your pallas kernel here")


# speedup vs baseline: 3.0947x; 3.0947x over previous
import functools

import jax
import jax.numpy as jnp
from jax import lax
from jax.experimental import pallas as pl
from jax.experimental.pallas import tpu as pltpu

_B, _D, _E, _K = 32768, 4096, 64, 8
_BT = 256
_NB = _B // _BT


def _router_body(x_ref, w_ref, b_ref, gw_ref, tki_ref, tkw_ref, lb_ref,
                 sp_ref, acc_ref, tks_ref):
    i = pl.program_id(0)

    logits = jnp.dot(x_ref[...], w_ref[...],
                     preferred_element_type=jnp.float32) + b_ref[...]
    m = jnp.max(logits, axis=1, keepdims=True)
    e = jnp.exp(logits - m)
    s = jnp.sum(e, axis=1, keepdims=True)
    gw = e / s
    gw_ref[...] = gw

    iota = lax.broadcasted_iota(jnp.int32, (_BT, _E), 1)
    g = gw
    w_cols = []
    i_cols = []
    for _ in range(_K):
        mx = jnp.max(g, axis=1, keepdims=True)
        idx = jnp.min(jnp.where(g == mx, iota, _E), axis=1, keepdims=True)
        w_cols.append(mx)
        i_cols.append(idx)
        g = jnp.where(iota == idx, -1.0, g)
    tkw = jnp.concatenate(w_cols, axis=1)
    tki = jnp.concatenate(i_cols, axis=1)
    tkw_ref[...] = tkw
    tki_ref[...] = tki

    sel = (g < 0.0).astype(jnp.float32)
    imp_blk = jnp.sum(gw, axis=0, keepdims=True)
    cnt_blk = jnp.sum(sel, axis=0, keepdims=True)
    blk = jnp.concatenate([imp_blk, cnt_blk], axis=0)

    @pl.when(i == 0)
    def _init():
        acc_ref[...] = blk
        tks_ref[0, 0] = jnp.sum(tkw)

    @pl.when(i > 0)
    def _accum():
        acc_ref[...] += blk
        tks_ref[0, 0] += jnp.sum(tkw)

    @pl.when(i == _NB - 1)
    def _finalize():
        acc = acc_ref[...]
        imp = acc[0:1, :] * (1.0 / _B)
        load = acc[1:2, :] * (1.0 / _B)
        lb_ref[0, 0] = _E * jnp.sum(imp * load)
        sp_ref[0, 0] = 1.0 - tks_ref[0, 0] * (1.0 / _B)


@functools.partial(jax.jit, static_argnames=())
def kernel(x, W, b):
    b2 = b.reshape(1, _E)
    out_shape = (
        jax.ShapeDtypeStruct((_B, _E), jnp.float32),
        jax.ShapeDtypeStruct((_B, _K), jnp.int32),
        jax.ShapeDtypeStruct((_B, _K), jnp.float32),
        jax.ShapeDtypeStruct((1, 1), jnp.float32),
        jax.ShapeDtypeStruct((1, 1), jnp.float32),
    )
    grid = (_NB,)
    gw, tki, tkw, lb, sp = pl.pallas_call(
        _router_body,
        grid=grid,
        in_specs=[
            pl.BlockSpec((_BT, _D), lambda i: (i, 0)),
            pl.BlockSpec((_D, _E), lambda i: (0, 0)),
            pl.BlockSpec((1, _E), lambda i: (0, 0)),
        ],
        out_specs=(
            pl.BlockSpec((_BT, _E), lambda i: (i, 0)),
            pl.BlockSpec((_BT, _K), lambda i: (i, 0)),
            pl.BlockSpec((_BT, _K), lambda i: (i, 0)),
            pl.BlockSpec(memory_space=pltpu.SMEM),
            pl.BlockSpec(memory_space=pltpu.SMEM),
        ),
        out_shape=out_shape,
        scratch_shapes=[
            pltpu.VMEM((2, _E), jnp.float32),
            pltpu.SMEM((1, 1), jnp.float32),
        ],
    )(x, W, b2)
    return (gw, tki, tkw, lb.reshape(()), sp.reshape(()))


# speedup vs baseline: 4.7288x; 1.5280x over previous
import functools

import jax
import jax.numpy as jnp
from jax import lax
from jax.experimental import pallas as pl
from jax.experimental.pallas import tpu as pltpu

_B, _D, _E, _K = 32768, 4096, 64, 8
_BT = 256
_NB = _B // _BT


def _router_body(x_ref, w_ref, b_ref, gw_ref, tki_ref, tkw_ref, lb_ref,
                 sp_ref, imp_ref, cnt_ref, tks_ref):
    i = pl.program_id(0)

    logits = jnp.dot(x_ref[...], w_ref[...],
                     preferred_element_type=jnp.float32) + b_ref[...]
    lt = logits.T
    m = jnp.max(lt, axis=0, keepdims=True)
    e = jnp.exp(lt - m)
    s = jnp.sum(e, axis=0, keepdims=True)
    gwt = e * (1.0 / s)
    gw_ref[...] = gwt.T

    iota = lax.broadcasted_iota(jnp.int32, (_E, _BT), 0)
    g = gwt
    w_rows = []
    i_rows = []
    for _ in range(_K):
        mx = jnp.max(g, axis=0, keepdims=True)
        idx = jnp.min(jnp.where(g == mx, iota, _E), axis=0, keepdims=True)
        w_rows.append(mx)
        i_rows.append(idx)
        g = jnp.where(iota == idx, -1.0, g)
    tkw_t = jnp.concatenate(w_rows, axis=0)
    tki_t = jnp.concatenate(i_rows, axis=0)
    tkw_ref[...] = tkw_t.T
    tki_ref[...] = tki_t.T

    sel = (g < 0.0).astype(jnp.float32)

    @pl.when(i == 0)
    def _init():
        imp_ref[...] = gwt
        cnt_ref[...] = sel
        tks_ref[...] = jnp.sum(tkw_t, axis=0, keepdims=True)

    @pl.when(i > 0)
    def _accum():
        imp_ref[...] += gwt
        cnt_ref[...] += sel
        tks_ref[...] += jnp.sum(tkw_t, axis=0, keepdims=True)

    @pl.when(i == _NB - 1)
    def _finalize():
        imp_col = jnp.sum(imp_ref[...], axis=1, keepdims=True)
        cnt_col = jnp.sum(cnt_ref[...], axis=1, keepdims=True)
        lb_ref[0, 0] = (_E / (_B * float(_B))) * jnp.sum(imp_col * cnt_col)
        sp_ref[0, 0] = 1.0 - jnp.sum(tks_ref[...]) * (1.0 / _B)


@functools.partial(jax.jit, static_argnames=())
def kernel(x, W, b):
    b2 = b.reshape(1, _E)
    out_shape = (
        jax.ShapeDtypeStruct((_B, _E), jnp.float32),
        jax.ShapeDtypeStruct((_B, _K), jnp.int32),
        jax.ShapeDtypeStruct((_B, _K), jnp.float32),
        jax.ShapeDtypeStruct((1, 1), jnp.float32),
        jax.ShapeDtypeStruct((1, 1), jnp.float32),
    )
    grid = (_NB,)
    gw, tki, tkw, lb, sp = pl.pallas_call(
        _router_body,
        grid=grid,
        in_specs=[
            pl.BlockSpec((_BT, _D), lambda i: (i, 0)),
            pl.BlockSpec((_D, _E), lambda i: (0, 0)),
            pl.BlockSpec((1, _E), lambda i: (0, 0)),
        ],
        out_specs=(
            pl.BlockSpec((_BT, _E), lambda i: (i, 0)),
            pl.BlockSpec((_BT, _K), lambda i: (i, 0)),
            pl.BlockSpec((_BT, _K), lambda i: (i, 0)),
            pl.BlockSpec(memory_space=pltpu.SMEM),
            pl.BlockSpec(memory_space=pltpu.SMEM),
        ),
        out_shape=out_shape,
        scratch_shapes=[
            pltpu.VMEM((_E, _BT), jnp.float32),
            pltpu.VMEM((_E, _BT), jnp.float32),
            pltpu.VMEM((1, _BT), jnp.float32),
        ],
    )(x, W, b2)
    return (gw, tki, tkw, lb.reshape(()), sp.reshape(()))


# speedup vs baseline: 5.7712x; 1.2204x over previous
import functools

import jax
import jax.numpy as jnp
from jax import lax
from jax.experimental import pallas as pl
from jax.experimental.pallas import tpu as pltpu

_B, _D, _E, _K = 32768, 4096, 64, 8
_BT = 512
_NB = _B // _BT


def _router_body(x_ref, w_ref, b_ref, gw_ref, tki_ref, tkw_ref, lb_ref,
                 sp_ref, imp_ref, cnt_ref, tks_ref):
    i = pl.program_id(0)

    logits = jnp.dot(x_ref[...], w_ref[...],
                     preferred_element_type=jnp.float32) + b_ref[...]
    lt = logits.T
    m = jnp.max(lt, axis=0, keepdims=True)
    e = jnp.exp(lt - m)
    s = jnp.sum(e, axis=0, keepdims=True)
    gwt = e * (1.0 / s)
    gw_ref[...] = gwt.T

    iota = lax.broadcasted_iota(jnp.int32, (_E, _BT), 0)
    g = gwt
    w_rows = []
    i_rows = []
    for _ in range(_K):
        mx = jnp.max(g, axis=0, keepdims=True)
        idx = jnp.min(jnp.where(g == mx, iota, _E), axis=0, keepdims=True)
        w_rows.append(mx)
        i_rows.append(idx)
        g = jnp.where(iota == idx, -1.0, g)
    tkw_t = jnp.concatenate(w_rows, axis=0)
    tki_t = jnp.concatenate(i_rows, axis=0)
    tkw_ref[...] = tkw_t.T
    tki_ref[...] = tki_t.T

    sel = (g < 0.0).astype(jnp.float32)

    @pl.when(i == 0)
    def _init():
        imp_ref[...] = gwt
        cnt_ref[...] = sel
        tks_ref[...] = jnp.sum(tkw_t, axis=0, keepdims=True)

    @pl.when(i > 0)
    def _accum():
        imp_ref[...] += gwt
        cnt_ref[...] += sel
        tks_ref[...] += jnp.sum(tkw_t, axis=0, keepdims=True)

    @pl.when(i == _NB - 1)
    def _finalize():
        imp_col = jnp.sum(imp_ref[...], axis=1, keepdims=True)
        cnt_col = jnp.sum(cnt_ref[...], axis=1, keepdims=True)
        lb_ref[0, 0] = (_E / (_B * float(_B))) * jnp.sum(imp_col * cnt_col)
        sp_ref[0, 0] = 1.0 - jnp.sum(tks_ref[...]) * (1.0 / _B)


@functools.partial(jax.jit, static_argnames=())
def kernel(x, W, b):
    b2 = b.reshape(1, _E)
    out_shape = (
        jax.ShapeDtypeStruct((_B, _E), jnp.float32),
        jax.ShapeDtypeStruct((_B, _K), jnp.int32),
        jax.ShapeDtypeStruct((_B, _K), jnp.float32),
        jax.ShapeDtypeStruct((1, 1), jnp.float32),
        jax.ShapeDtypeStruct((1, 1), jnp.float32),
    )
    grid = (_NB,)
    gw, tki, tkw, lb, sp = pl.pallas_call(
        _router_body,
        grid=grid,
        in_specs=[
            pl.BlockSpec((_BT, _D), lambda i: (i, 0)),
            pl.BlockSpec((_D, _E), lambda i: (0, 0)),
            pl.BlockSpec((1, _E), lambda i: (0, 0)),
        ],
        out_specs=(
            pl.BlockSpec((_BT, _E), lambda i: (i, 0)),
            pl.BlockSpec((_BT, _K), lambda i: (i, 0)),
            pl.BlockSpec((_BT, _K), lambda i: (i, 0)),
            pl.BlockSpec(memory_space=pltpu.SMEM),
            pl.BlockSpec(memory_space=pltpu.SMEM),
        ),
        out_shape=out_shape,
        scratch_shapes=[
            pltpu.VMEM((_E, _BT), jnp.float32),
            pltpu.VMEM((_E, _BT), jnp.float32),
            pltpu.VMEM((1, _BT), jnp.float32),
        ],
    )(x, W, b2)
    return (gw, tki, tkw, lb.reshape(()), sp.reshape(()))


# speedup vs baseline: 6.1387x; 1.0637x over previous
import functools

import jax
import jax.numpy as jnp
from jax import lax
from jax.experimental import pallas as pl
from jax.experimental.pallas import tpu as pltpu

_B, _D, _E, _K = 32768, 4096, 64, 8
_BT = 1024
_NB = _B // _BT


def _router_body(x_ref, w_ref, b_ref, gw_ref, tki_ref, tkw_ref, lb_ref,
                 sp_ref, imp_ref, cnt_ref, tks_ref):
    i = pl.program_id(0)

    logits = jnp.dot(x_ref[...], w_ref[...],
                     preferred_element_type=jnp.float32) + b_ref[...]
    lt = logits.T
    m = jnp.max(lt, axis=0, keepdims=True)
    e = jnp.exp(lt - m)
    s = jnp.sum(e, axis=0, keepdims=True)
    gwt = e * (1.0 / s)
    gw_ref[...] = gwt.T

    iota = lax.broadcasted_iota(jnp.int32, (_E, _BT), 0)
    g = gwt
    w_rows = []
    i_rows = []
    for _ in range(_K):
        mx = jnp.max(g, axis=0, keepdims=True)
        idx = jnp.min(jnp.where(g == mx, iota, _E), axis=0, keepdims=True)
        w_rows.append(mx)
        i_rows.append(idx)
        g = jnp.where(iota == idx, -1.0, g)
    tkw_t = jnp.concatenate(w_rows, axis=0)
    tki_t = jnp.concatenate(i_rows, axis=0)
    tkw_ref[...] = tkw_t.T
    tki_ref[...] = tki_t.T

    sel = (g < 0.0).astype(jnp.float32)

    @pl.when(i == 0)
    def _init():
        imp_ref[...] = gwt
        cnt_ref[...] = sel
        tks_ref[...] = jnp.sum(tkw_t, axis=0, keepdims=True)

    @pl.when(i > 0)
    def _accum():
        imp_ref[...] += gwt
        cnt_ref[...] += sel
        tks_ref[...] += jnp.sum(tkw_t, axis=0, keepdims=True)

    @pl.when(i == _NB - 1)
    def _finalize():
        imp_col = jnp.sum(imp_ref[...], axis=1, keepdims=True)
        cnt_col = jnp.sum(cnt_ref[...], axis=1, keepdims=True)
        lb_ref[0, 0] = (_E / (_B * float(_B))) * jnp.sum(imp_col * cnt_col)
        sp_ref[0, 0] = 1.0 - jnp.sum(tks_ref[...]) * (1.0 / _B)


@functools.partial(jax.jit, static_argnames=())
def kernel(x, W, b):
    b2 = b.reshape(1, _E)
    out_shape = (
        jax.ShapeDtypeStruct((_B, _E), jnp.float32),
        jax.ShapeDtypeStruct((_B, _K), jnp.int32),
        jax.ShapeDtypeStruct((_B, _K), jnp.float32),
        jax.ShapeDtypeStruct((1, 1), jnp.float32),
        jax.ShapeDtypeStruct((1, 1), jnp.float32),
    )
    grid = (_NB,)
    gw, tki, tkw, lb, sp = pl.pallas_call(
        _router_body,
        grid=grid,
        in_specs=[
            pl.BlockSpec((_BT, _D), lambda i: (i, 0)),
            pl.BlockSpec((_D, _E), lambda i: (0, 0)),
            pl.BlockSpec((1, _E), lambda i: (0, 0)),
        ],
        out_specs=(
            pl.BlockSpec((_BT, _E), lambda i: (i, 0)),
            pl.BlockSpec((_BT, _K), lambda i: (i, 0)),
            pl.BlockSpec((_BT, _K), lambda i: (i, 0)),
            pl.BlockSpec(memory_space=pltpu.SMEM),
            pl.BlockSpec(memory_space=pltpu.SMEM),
        ),
        out_shape=out_shape,
        scratch_shapes=[
            pltpu.VMEM((_E, _BT), jnp.float32),
            pltpu.VMEM((_E, _BT), jnp.float32),
            pltpu.VMEM((1, _BT), jnp.float32),
        ],
    )(x, W, b2)
    return (gw, tki, tkw, lb.reshape(()), sp.reshape(()))
